# baseline (speedup 1.0000x reference)
import jax
import jax.numpy as jnp
from jax import lax
from jax.experimental import pallas as pl
from jax.experimental.pallas import tpu as pltpu
from jax.experimental.pallas import tpu_sc as plsc

NUM_EMBEDDINGS = 1000000
EMBEDDING_DIM = 32
BATCH = 16384
N_FIELDS = 26

NC = 2
NS = 16
NW = NC * NS

LT = BATCH // 128
UNITS = N_FIELDS * LT
PER_W = UNITS // NW
UPC = 8
NCHUNK = PER_W // UPC
CLOOK = UPC * 128

WPR = EMBEDDING_DIM // 2
STBLK = UPC * 1024

_CP = pltpu.CompilerParams(
    needs_layout_passes=False, use_tc_tiling_on_sc=False)
_MESH = dict(core_axis_name="c", subcore_axis_name="s")


def _gather_body(idx_hbm, tab_hbm, scale_hbm, out_hbm,
                 idx_a, idx_b, idx_c, rows_a, rows_b, rows_c,
                 out_a, out_b, scale_v,
                 sem_i0, sem_i1, sem_i2, sem_g0, sem_g1, sem_g2,
                 sem_o0, sem_o1):
    wid = lax.axis_index("s") * NC + lax.axis_index("c")
    ubase = wid * PER_W

    pltpu.sync_copy(scale_hbm, scale_v)
    scale = scale_v[...]

    lane = lax.iota(jnp.int32, 16)
    e_even = lane * 2
    evec_e = (e_even // 8) * STBLK + (e_even % 8) * 128
    e_odd = e_even + 1
    evec_o = (e_odd // 8) * STBLK + (e_odd % 8) * 128

    idx_v = (idx_a, idx_b, idx_c)
    rows_v = (rows_a, rows_b, rows_c)
    out_v = (out_a, out_b)
    sem_i = (sem_i0, sem_i1, sem_i2)
    sem_g = (sem_g0, sem_g1, sem_g2)
    sem_o = (sem_o0, sem_o1)

    def flt(ci):
        u0 = ubase + ci * UPC
        return u0 // LT, u0 % LT

    def fire_idx(ci):
        f, lt0 = flt(ci)
        return pltpu.async_copy(
            idx_hbm.at[f, pl.ds(lt0 * 128, CLOOK)], idx_v[ci % 3],
            sem_i[ci % 3])

    def fire_gather(ci):
        return pltpu.async_copy(
            tab_hbm.at[idx_v[ci % 3]], rows_v[ci % 3], sem_g[ci % 3])

    def fire_out(ci):
        f, lt0 = flt(ci)
        return [pltpu.async_copy(
            out_v[ci % 2].at[pl.ds(st * STBLK, STBLK)],
            out_hbm.at[f, st, pl.ds(lt0 * 1024, STBLK)],
            sem_o[ci % 2]) for st in range(4)]

    def compute(ci):
        rows = rows_v[ci % 3]
        out = out_v[ci % 2]

        mask = jnp.int32(-1879056384)

        @plsc.parallel_loop(0, CLOOK, unroll=8)
        def row_body(j):
            w = plsc.bitcast(rows[j, :], jnp.int32)
            lo = ((w << 16) >> 3) & mask
            hi = (w >> 3) & mask
            ev = lax.bitcast_convert_type(lo, jnp.float32) * scale
            od = lax.bitcast_convert_type(hi, jnp.float32) * scale
            base = ((j >> 7) << 10) | (j & 127)
            plsc.store_scatter(out, [base + evec_e], ev)
            plsc.store_scatter(out, [base + evec_o], od)

    icp = [None] * NCHUNK
    gcp = [None] * NCHUNK
    ocp = [None] * NCHUNK
    icp[0] = fire_idx(0)
    icp[1] = fire_idx(1)
    icp[0].wait()
    gcp[0] = fire_gather(0)
    icp[1].wait()
    gcp[1] = fire_gather(1)
    icp[2] = fire_idx(2)
    for ci in range(NCHUNK):
        if ci + 2 < NCHUNK:
            icp[ci + 2].wait()
            gcp[ci + 2] = fire_gather(ci + 2)
        gcp[ci].wait()
        if ci >= 2:
            for cp in ocp[ci - 2]:
                cp.wait()
        compute(ci)
        ocp[ci] = fire_out(ci)
        if ci + 3 < NCHUNK:
            icp[ci + 3] = fire_idx(ci + 3)
    for cp in ocp[NCHUNK - 2] + ocp[NCHUNK - 1]:
        cp.wait()


@jax.jit
def _run(xt, tab, scale16):
    gather = pl.kernel(
        _gather_body,
        mesh=plsc.VectorSubcoreMesh(**_MESH),
        compiler_params=_CP,
        out_type=jax.ShapeDtypeStruct(
            (N_FIELDS, 4, LT * 1024), jnp.float32),
        scratch_types=[
            pltpu.VMEM((CLOOK,), jnp.int32),
            pltpu.VMEM((CLOOK,), jnp.int32),
            pltpu.VMEM((CLOOK,), jnp.int32),
            pltpu.VMEM((CLOOK, EMBEDDING_DIM), jnp.float16),
            pltpu.VMEM((CLOOK, EMBEDDING_DIM), jnp.float16),
            pltpu.VMEM((CLOOK, EMBEDDING_DIM), jnp.float16),
            pltpu.VMEM((4 * STBLK,), jnp.float32),
            pltpu.VMEM((4 * STBLK,), jnp.float32),
            pltpu.VMEM((16,), jnp.float32),
        ] + [pltpu.SemaphoreType.DMA] * 8,
    )
    return gather(xt, tab, scale16)


def kernel(x, weight_quant, c):
    xt = x.T
    scale = jnp.float32(2.0 ** 112) / c
    scale16 = jnp.broadcast_to(scale, (16,))
    out = _run(xt, weight_quant, scale16)
    out = out.reshape(N_FIELDS, 4, LT, 8, 128)
    out = out.transpose(2, 4, 0, 1, 3)
    return out.reshape(BATCH, N_FIELDS, EMBEDDING_DIM)
